# Initial kernel scaffold; baseline (speedup 1.0000x reference)
import jax
import jax.numpy as jnp
from jax.experimental import pallas as pl


def kernel(feat, edge_index, edge_weight, edge_mask, W_self, b_self, W_neigh, b_neigh):
    raise NotImplementedError("write your pallas kernel here")


# speedup vs baseline: 3.4854x; 3.4854x over previous
import jax
import jax.numpy as jnp
from jax import lax
from jax.experimental import pallas as pl
from jax.experimental.pallas import tpu as pltpu
from jax.experimental.pallas import tpu_sc as plsc

N_NODES = 10000
N_EDGES = 320000
D = 128
NC = 2
NS = 16
NW = NC * NS
EPW = N_EDGES // NW
B = 80
NB = EPW // B
ROWS_PER_TILE = 632
ACC_ROWS = NS * ROWS_PER_TILE
DEG_ROWS = 80


def _sc_body(src_hbm, dst_hbm, ew_hbm, em_hbm, feat_hbm,
             out_hbm, outd_hbm,
             acc, accd, src_v, dst_v, dstd_v, w_v, m_v, gbuf, sbuf, dbuf, sem):
    c = lax.axis_index("c")
    s = lax.axis_index("s")
    wid = c * NS + s

    zeros16 = jnp.zeros((16,), jnp.float32)
    ones16 = jnp.ones((16,), jnp.float32)
    iota16 = lax.broadcasted_iota(jnp.int32, (16,), 0)

    def _zero_row(r, carry):
        for cc in range(D // 16):
            sbuf[r, pl.ds(cc * 16, 16)] = zeros16
            dbuf[r, pl.ds(cc * 16, 16)] = zeros16
        return carry
    lax.fori_loop(0, B, _zero_row, 0)

    tile_base = s * ROWS_PER_TILE
    for k in range(7):
        pltpu.sync_copy(sbuf, acc.at[pl.ds(tile_base + k * B, B)])
    pltpu.sync_copy(sbuf.at[pl.ds(0, 72)], acc.at[pl.ds(tile_base + 560, 72)])

    @pl.when(s == 0)
    def _zero_deg():
        pltpu.sync_copy(sbuf, accd)

    plsc.subcore_barrier()

    def _batch(i, carry):
        base = wid * EPW + i * B
        pltpu.sync_copy(src_hbm.at[pl.ds(base, B)], src_v)
        pltpu.sync_copy(dst_hbm.at[pl.ds(base, B)], dst_v)
        pltpu.sync_copy(ew_hbm.at[pl.ds(base, B)], w_v)
        pltpu.sync_copy(em_hbm.at[pl.ds(base, B)], m_v)
        pltpu.async_copy(feat_hbm.at[src_v], gbuf, sem).wait()
        for j in range(B // 16):
            sl = pl.ds(j * 16, 16)
            w_v[sl] = w_v[sl] * m_v[sl]
            dstd_v[sl] = lax.shift_right_logical(dst_v[sl], 7)

        def _group(g, gcarry):
            wv16 = w_v[pl.ds(g * 16, 16)]
            for k in range(16):
                wb = jnp.full((16,), wv16[k], jnp.float32)
                r = g * 16 + k
                for cc in range(D // 16):
                    sl = pl.ds(cc * 16, 16)
                    sbuf[r, sl] = gbuf[r, sl] * wb
            return gcarry
        lax.fori_loop(0, B // 16, _group, 0)
        pltpu.sync_copy(sbuf, acc.at[dst_v], add=True)

        def _dgroup(g, gcarry):
            dv16 = dst_v[pl.ds(g * 16, 16)]
            for k in range(16):
                col = lax.bitwise_and(dv16[k], 127)
                off = lax.bitwise_and(col, 112)
                lane = lax.bitwise_and(col, 15)
                d = iota16 - jnp.full((16,), lane, jnp.int32)
                oh = (1 - jnp.minimum(jnp.abs(d), 1)).astype(jnp.float32)
                dbuf[g * 16 + k, pl.ds(off, 16)] = oh
            return gcarry
        lax.fori_loop(0, B // 16, _dgroup, 0)
        pltpu.sync_copy(dbuf, accd.at[dstd_v], add=True)

        def _dclear(g, gcarry):
            dv16 = dst_v[pl.ds(g * 16, 16)]
            for k in range(16):
                off = lax.bitwise_and(dv16[k], 112)
                dbuf[g * 16 + k, pl.ds(off, 16)] = zeros16
            return gcarry
        lax.fori_loop(0, B // 16, _dclear, 0)
        return carry

    lax.fori_loop(0, NB, _batch, 0)

    plsc.subcore_barrier()

    pltpu.sync_copy(acc.at[pl.ds(tile_base, ROWS_PER_TILE)],
                    out_hbm.at[c, pl.ds(tile_base, ROWS_PER_TILE)])

    @pl.when(s == 0)
    def _copy_deg():
        pltpu.sync_copy(accd, outd_hbm.at[c])


def _sc_aggregate(src, dst, ew, em, feat):
    mesh = plsc.VectorSubcoreMesh(core_axis_name="c", subcore_axis_name="s")
    k = pl.kernel(
        _sc_body,
        mesh=mesh,
        out_type=(
            jax.ShapeDtypeStruct((NC, ACC_ROWS, D), jnp.float32),
            jax.ShapeDtypeStruct((NC, DEG_ROWS, D), jnp.float32),
        ),
        scratch_types=[
            pltpu.VMEM_SHARED((ACC_ROWS, D), jnp.float32),
            pltpu.VMEM_SHARED((DEG_ROWS, D), jnp.float32),
            pltpu.VMEM((B,), jnp.int32),
            pltpu.VMEM((B,), jnp.int32),
            pltpu.VMEM((B,), jnp.int32),
            pltpu.VMEM((B,), jnp.float32),
            pltpu.VMEM((B,), jnp.float32),
            pltpu.VMEM((B, D), jnp.float32),
            pltpu.VMEM((B, D), jnp.float32),
            pltpu.VMEM((B, D), jnp.float32),
            pltpu.SemaphoreType.DMA,
        ],
    )
    return k(src, dst, ew, em, feat)


BM = 1280


def _tc_finish_body(acc_ref, deg_ref, feat_ref, wst_ref, wnt_ref, b_ref, out_ref):
    msg = acc_ref[0] + acc_ref[1]
    deg = (deg_ref[0] + deg_ref[1]).reshape(BM, 1)
    h = msg / jnp.maximum(deg, 1.0)
    out_ref[...] = (
        jnp.dot(feat_ref[...], wst_ref[...], preferred_element_type=jnp.float32)
        + jnp.dot(h, wnt_ref[...], preferred_element_type=jnp.float32)
        + b_ref[...]
    )


def _tc_finish(acc, deg, feat, wst, wnt, b):
    grid = ((N_NODES + BM - 1) // BM,)
    return pl.pallas_call(
        _tc_finish_body,
        grid=grid,
        in_specs=[
            pl.BlockSpec((NC, BM, D), lambda i: (0, i, 0)),
            pl.BlockSpec((NC, BM), lambda i: (0, i)),
            pl.BlockSpec((BM, D), lambda i: (i, 0)),
            pl.BlockSpec((D, D), lambda i: (0, 0)),
            pl.BlockSpec((D, D), lambda i: (0, 0)),
            pl.BlockSpec((1, D), lambda i: (0, 0)),
        ],
        out_specs=pl.BlockSpec((BM, D), lambda i: (i, 0)),
        out_shape=jax.ShapeDtypeStruct((N_NODES, D), jnp.float32),
    )(acc, deg, feat, wst, wnt, b)


def kernel(feat, edge_index, edge_weight, edge_mask,
           W_self, b_self, W_neigh, b_neigh):
    src = edge_index[0].astype(jnp.int32)
    dst = edge_index[1].astype(jnp.int32)
    ew = edge_weight.reshape(-1)
    em = edge_mask.reshape(-1)
    acc, deg = _sc_aggregate(src, dst, ew, em, feat)
    deg = deg.reshape(NC, DEG_ROWS * D)
    b = (b_self + b_neigh).reshape(1, D)
    return _tc_finish(acc, deg, feat, W_self.T, W_neigh.T, b)


# speedup vs baseline: 6.0256x; 1.7288x over previous
import jax
import jax.numpy as jnp
from jax import lax
from jax.experimental import pallas as pl
from jax.experimental.pallas import tpu as pltpu
from jax.experimental.pallas import tpu_sc as plsc

N_NODES = 10000
N_EDGES = 320000
D = 128
NC = 2
NS = 16
NW = NC * NS
EPW = N_EDGES // NW
B = 80
NB = EPW // B
ROWS_PER_TILE = 632
ACC_ROWS = NS * ROWS_PER_TILE
DEG_ROWS = 80


def _sc_body(src_hbm, dst_hbm, ew_hbm, em_hbm, feat_hbm,
             out_hbm, outd_hbm,
             acc, accd,
             srcb0, srcb1, dstb0, dstb1, ewb0, ewb1, emb0, emb1,
             gbuf0, gbuf1, sbuf, dbuf, dstd_v, idmat,
             gsem0, gsem1, isem0, isem1, ssem, tsem):
    c = lax.axis_index("c")
    s = lax.axis_index("s")
    wid = c * NS + s
    ebase = wid * EPW

    zeros16 = jnp.zeros((16,), jnp.float32)
    iota16 = lax.broadcasted_iota(jnp.int32, (16,), 0)
    gbufs = (gbuf0, gbuf1)
    gsems = (gsem0, gsem1)
    isems = (isem0, isem1)
    srcbs = (srcb0, srcb1)
    dstbs = (dstb0, dstb1)
    ewbs = (ewb0, ewb1)
    embs = (emb0, emb1)

    def _mk_id(k, carry):
        d = iota16 - jnp.full((16,), k, jnp.int32)
        idmat[k, pl.ds(0, 16)] = (1 - jnp.minimum(jnp.abs(d), 1)).astype(
            jnp.float32)
        return carry
    lax.fori_loop(0, 16, _mk_id, 0)

    def _zero_row(r, carry):
        for cc in range(D // 16):
            sl = pl.ds(cc * 16, 16)
            sbuf[r, sl] = zeros16
            dbuf[r, sl] = zeros16
        return carry
    lax.fori_loop(0, B, _zero_row, 0)

    tile_base = s * ROWS_PER_TILE
    for k in range(7):
        pltpu.sync_copy(sbuf, acc.at[pl.ds(tile_base + k * B, B)])
    pltpu.sync_copy(sbuf.at[pl.ds(0, 72)], acc.at[pl.ds(tile_base + 560, 72)])

    @pl.when(s == 0)
    def _zero_deg():
        pltpu.sync_copy(sbuf, accd)

    plsc.subcore_barrier()

    def _issue_idx(i, b):
        base = ebase + i * B
        pltpu.async_copy(src_hbm.at[pl.ds(base, B)], srcbs[b], isems[b])
        pltpu.async_copy(dst_hbm.at[pl.ds(base, B)], dstbs[b], isems[b])
        pltpu.async_copy(ew_hbm.at[pl.ds(base, B)], ewbs[b], isems[b])
        pltpu.async_copy(em_hbm.at[pl.ds(base, B)], embs[b], isems[b])

    def _drain_idx(i, b):
        base = ebase + i * B
        pltpu.make_async_copy(src_hbm.at[pl.ds(base, B)], srcbs[b], isems[b]).wait()
        pltpu.make_async_copy(dst_hbm.at[pl.ds(base, B)], dstbs[b], isems[b]).wait()
        pltpu.make_async_copy(ew_hbm.at[pl.ds(base, B)], ewbs[b], isems[b]).wait()
        pltpu.make_async_copy(em_hbm.at[pl.ds(base, B)], embs[b], isems[b]).wait()

    def _compute(b):
        gbuf = gbufs[b]

        def _group(g, gcarry):
            sl16 = pl.ds(g * 16, 16)
            wv16 = ewbs[b][sl16] * embs[b][sl16]
            dv16 = dstbs[b][sl16]
            dstd_v[sl16] = lax.shift_right_logical(dv16, 7)
            for k in range(16):
                wb = jnp.full((16,), wv16[k], jnp.float32)
                r = g * 16 + k
                for cc in range(D // 16):
                    sl = pl.ds(cc * 16, 16)
                    sbuf[r, sl] = gbuf[r, sl] * wb
                col = lax.bitwise_and(dv16[k], 127)
                off = lax.bitwise_and(col, 112)
                lane = lax.bitwise_and(col, 15)
                dbuf[r, pl.ds(off, 16)] = idmat[lane, pl.ds(0, 16)]
            return gcarry
        lax.fori_loop(0, B // 16, _group, 0)

    def _clear_deg(b):
        def _group(g, gcarry):
            dv16 = dstbs[b][pl.ds(g * 16, 16)]
            for k in range(16):
                off = lax.bitwise_and(dv16[k], 112)
                dbuf[g * 16 + k, pl.ds(off, 16)] = zeros16
            return gcarry
        lax.fori_loop(0, B // 16, _group, 0)

    def _batch(i, b, last=False):
        pltpu.make_async_copy(feat_hbm.at[srcbs[b]], gbufs[b], gsems[b]).wait()
        if not last:
            _drain_idx(i + 1, 1 - b)
            pltpu.async_copy(feat_hbm.at[srcbs[1 - b]], gbufs[1 - b],
                             gsems[1 - b])
        _compute(b)
        sc = pltpu.async_copy(sbuf, acc.at[dstbs[b]], ssem, add=True)
        tc_ = pltpu.async_copy(dbuf, accd.at[dstd_v], tsem, add=True)
        sc.wait()
        tc_.wait()
        _clear_deg(b)
        if not last:
            @pl.when(i + 2 < NB)
            def _():
                _issue_idx(i + 2, b)

    _issue_idx(0, 0)
    _drain_idx(0, 0)
    _issue_idx(1, 1)
    pltpu.async_copy(feat_hbm.at[srcb0], gbuf0, gsem0)

    def _pair(p, carry):
        _batch(2 * p, 0)
        _batch(2 * p + 1, 1)
        return carry

    lax.fori_loop(0, NB // 2, _pair, 0)
    _batch(NB - 1, 0, last=True)

    plsc.subcore_barrier()

    pltpu.sync_copy(acc.at[pl.ds(tile_base, ROWS_PER_TILE)],
                    out_hbm.at[c, pl.ds(tile_base, ROWS_PER_TILE)])

    @pl.when(s == 0)
    def _copy_deg():
        pltpu.sync_copy(accd, outd_hbm.at[c])


def _sc_aggregate(src, dst, ew, em, feat):
    mesh = plsc.VectorSubcoreMesh(core_axis_name="c", subcore_axis_name="s")
    k = pl.kernel(
        _sc_body,
        mesh=mesh,
        out_type=(
            jax.ShapeDtypeStruct((NC, ACC_ROWS, D), jnp.float32),
            jax.ShapeDtypeStruct((NC, DEG_ROWS, D), jnp.float32),
        ),
        scratch_types=[
            pltpu.VMEM_SHARED((ACC_ROWS, D), jnp.float32),
            pltpu.VMEM_SHARED((DEG_ROWS, D), jnp.float32),
            pltpu.VMEM((B,), jnp.int32),
            pltpu.VMEM((B,), jnp.int32),
            pltpu.VMEM((B,), jnp.int32),
            pltpu.VMEM((B,), jnp.int32),
            pltpu.VMEM((B,), jnp.float32),
            pltpu.VMEM((B,), jnp.float32),
            pltpu.VMEM((B,), jnp.float32),
            pltpu.VMEM((B,), jnp.float32),
            pltpu.VMEM((B, D), jnp.float32),
            pltpu.VMEM((B, D), jnp.float32),
            pltpu.VMEM((B, D), jnp.float32),
            pltpu.VMEM((B, D), jnp.float32),
            pltpu.VMEM((B,), jnp.int32),
            pltpu.VMEM((16, 16), jnp.float32),
            pltpu.SemaphoreType.DMA,
            pltpu.SemaphoreType.DMA,
            pltpu.SemaphoreType.DMA,
            pltpu.SemaphoreType.DMA,
            pltpu.SemaphoreType.DMA,
            pltpu.SemaphoreType.DMA,
        ],
    )
    return k(src, dst, ew, em, feat)


BM = 1280


def _tc_finish_body(acc_ref, deg_ref, feat_ref, wst_ref, wnt_ref, b_ref, out_ref):
    msg = acc_ref[0] + acc_ref[1]
    deg = (deg_ref[0] + deg_ref[1]).reshape(BM, 1)
    h = msg / jnp.maximum(deg, 1.0)
    out_ref[...] = (
        jnp.dot(feat_ref[...], wst_ref[...], preferred_element_type=jnp.float32)
        + jnp.dot(h, wnt_ref[...], preferred_element_type=jnp.float32)
        + b_ref[...]
    )


def _tc_finish(acc, deg, feat, wst, wnt, b):
    grid = ((N_NODES + BM - 1) // BM,)
    return pl.pallas_call(
        _tc_finish_body,
        grid=grid,
        in_specs=[
            pl.BlockSpec((NC, BM, D), lambda i: (0, i, 0)),
            pl.BlockSpec((NC, BM), lambda i: (0, i)),
            pl.BlockSpec((BM, D), lambda i: (i, 0)),
            pl.BlockSpec((D, D), lambda i: (0, 0)),
            pl.BlockSpec((D, D), lambda i: (0, 0)),
            pl.BlockSpec((1, D), lambda i: (0, 0)),
        ],
        out_specs=pl.BlockSpec((BM, D), lambda i: (i, 0)),
        out_shape=jax.ShapeDtypeStruct((N_NODES, D), jnp.float32),
    )(acc, deg, feat, wst, wnt, b)


def kernel(feat, edge_index, edge_weight, edge_mask,
           W_self, b_self, W_neigh, b_neigh):
    src = edge_index[0].astype(jnp.int32)
    dst = edge_index[1].astype(jnp.int32)
    ew = edge_weight.reshape(-1)
    em = edge_mask.reshape(-1)
    acc, deg = _sc_aggregate(src, dst, ew, em, feat)
    deg = deg.reshape(NC, DEG_ROWS * D)
    b = (b_self + b_neigh).reshape(1, D)
    return _tc_finish(acc, deg, feat, W_self.T, W_neigh.T, b)


# speedup vs baseline: 6.8238x; 1.1325x over previous
import jax
import jax.numpy as jnp
from jax import lax
from jax.experimental import pallas as pl
from jax.experimental.pallas import tpu as pltpu
from jax.experimental.pallas import tpu_sc as plsc

N_NODES = 10000
N_EDGES = 320000
D = 128
NC = 2
NS = 16
NW = NC * NS
EPW = N_EDGES // NW
B = 80
NB = EPW // B
ROWS_PER_TILE = 632
ACC_ROWS = NS * ROWS_PER_TILE
DEG_ROWS = 80


def _sc_body(src_hbm, dst_hbm, ew_hbm, em_hbm, feat_hbm,
             out_hbm, outd_hbm,
             acc, accd,
             srcb0, srcb1, srcb2, dstb0, dstb1, dstb2,
             ewb0, ewb1, ewb2, emb0, emb1, emb2,
             gbuf0, gbuf1, dbuf0, dbuf1, dstd0, dstd1, idmat,
             gsem0, gsem1, isem0, isem1, isem2, ssem0, ssem1, tsem0, tsem1):
    c = lax.axis_index("c")
    s = lax.axis_index("s")
    wid = c * NS + s
    ebase = wid * EPW

    zeros16 = jnp.zeros((16,), jnp.float32)
    iota16 = lax.broadcasted_iota(jnp.int32, (16,), 0)
    gbufs = (gbuf0, gbuf1)
    gsems = (gsem0, gsem1)
    dbufs = (dbuf0, dbuf1)
    dstds = (dstd0, dstd1)
    ssems = (ssem0, ssem1)
    tsems = (tsem0, tsem1)
    isems = (isem0, isem1, isem2)
    srcbs = (srcb0, srcb1, srcb2)
    dstbs = (dstb0, dstb1, dstb2)
    ewbs = (ewb0, ewb1, ewb2)
    embs = (emb0, emb1, emb2)

    def _mk_id(k, carry):
        d = iota16 - jnp.full((16,), k, jnp.int32)
        idmat[k, pl.ds(0, 16)] = (1 - jnp.minimum(jnp.abs(d), 1)).astype(
            jnp.float32)
        return carry
    lax.fori_loop(0, 16, _mk_id, 0)

    def _zero_row(r, carry):
        for cc in range(D // 16):
            sl = pl.ds(cc * 16, 16)
            gbuf0[r, sl] = zeros16
            dbuf0[r, sl] = zeros16
            dbuf1[r, sl] = zeros16
        return carry
    lax.fori_loop(0, B, _zero_row, 0)

    tile_base = s * ROWS_PER_TILE
    for k in range(7):
        pltpu.sync_copy(gbuf0, acc.at[pl.ds(tile_base + k * B, B)])
    pltpu.sync_copy(gbuf0.at[pl.ds(0, 72)], acc.at[pl.ds(tile_base + 560, 72)])

    @pl.when(s == 0)
    def _zero_deg():
        pltpu.sync_copy(gbuf0, accd)

    plsc.subcore_barrier()

    def _issue_idx(i, b3):
        base = ebase + i * B
        pltpu.async_copy(src_hbm.at[pl.ds(base, B)], srcbs[b3], isems[b3])
        pltpu.async_copy(dst_hbm.at[pl.ds(base, B)], dstbs[b3], isems[b3])
        pltpu.async_copy(ew_hbm.at[pl.ds(base, B)], ewbs[b3], isems[b3])
        pltpu.async_copy(em_hbm.at[pl.ds(base, B)], embs[b3], isems[b3])

    def _drain_idx(i, b3):
        base = ebase + i * B
        pltpu.make_async_copy(src_hbm.at[pl.ds(base, B)], srcbs[b3], isems[b3]).wait()
        pltpu.make_async_copy(dst_hbm.at[pl.ds(base, B)], dstbs[b3], isems[b3]).wait()
        pltpu.make_async_copy(ew_hbm.at[pl.ds(base, B)], ewbs[b3], isems[b3]).wait()
        pltpu.make_async_copy(em_hbm.at[pl.ds(base, B)], embs[b3], isems[b3]).wait()

    def _wait_scatters(b2, b3):
        pltpu.make_async_copy(gbufs[b2], acc.at[dstbs[b3]], ssems[b2]).wait()
        pltpu.make_async_copy(dbufs[b2], accd.at[dstds[b2]], tsems[b2]).wait()

    def _compute(b2, b3):
        gbuf = gbufs[b2]
        dbuf = dbufs[b2]

        def _group(g, gcarry):
            sl16 = pl.ds(g * 16, 16)
            wv16 = ewbs[b3][sl16] * embs[b3][sl16]
            dv16 = dstbs[b3][sl16]
            dstds[b2][sl16] = lax.shift_right_logical(dv16, 7)
            for k in range(16):
                wb = jnp.full((16,), wv16[k], jnp.float32)
                r = g * 16 + k
                for cc in range(D // 16):
                    sl = pl.ds(cc * 16, 16)
                    gbuf[r, sl] = gbuf[r, sl] * wb
                col = lax.bitwise_and(dv16[k], 127)
                off = lax.bitwise_and(col, 112)
                lane = lax.bitwise_and(col, 15)
                dbuf[r, pl.ds(off, 16)] = idmat[lane, pl.ds(0, 16)]
            return gcarry
        lax.fori_loop(0, B // 16, _group, 0)

    def _clear_deg(b2, b3):
        def _group(g, gcarry):
            dv16 = dstbs[b3][pl.ds(g * 16, 16)]
            for k in range(16):
                off = lax.bitwise_and(dv16[k], 112)
                dbufs[b2][g * 16 + k, pl.ds(off, 16)] = zeros16
            return gcarry
        lax.fori_loop(0, B // 16, _group, 0)

    def _batch(i, k, first=False, last=False):
        b2, b3 = k % 2, k % 3
        pb2, pb3 = (k + 1) % 2, (k + 2) % 3
        pltpu.make_async_copy(feat_hbm.at[srcbs[b3]], gbufs[b2], gsems[b2]).wait()
        if not first:
            _wait_scatters(pb2, pb3)
            _clear_deg(pb2, pb3)
        if not last:
            _drain_idx(i + 1, (k + 1) % 3)
            pltpu.async_copy(feat_hbm.at[srcbs[(k + 1) % 3]], gbufs[pb2],
                             gsems[pb2])
        _compute(b2, b3)
        pltpu.async_copy(gbufs[b2], acc.at[dstbs[b3]], ssems[b2], add=True)
        pltpu.async_copy(dbufs[b2], accd.at[dstds[b2]], tsems[b2], add=True)
        if not last:
            @pl.when(i + 2 < NB)
            def _():
                _issue_idx(i + 2, pb3)

    _issue_idx(0, 0)
    _drain_idx(0, 0)
    _issue_idx(1, 1)
    pltpu.async_copy(feat_hbm.at[srcb0], gbuf0, gsem0)

    _batch(0, 0, first=True)
    for k in range(1, 6):
        _batch(k, k)

    def _six(p, carry):
        for k in range(6):
            _batch(6 * p + k, k)
        return carry
    lax.fori_loop(1, NB // 6, _six, 0)

    for k in range(4):
        _batch(120 + k, k)
    _batch(124, 4, last=True)
    _wait_scatters(0, 1)

    plsc.subcore_barrier()

    pltpu.sync_copy(acc.at[pl.ds(tile_base, ROWS_PER_TILE)],
                    out_hbm.at[c, pl.ds(tile_base, ROWS_PER_TILE)])

    @pl.when(s == 0)
    def _copy_deg():
        pltpu.sync_copy(accd, outd_hbm.at[c])


def _sc_aggregate(src, dst, ew, em, feat):
    mesh = plsc.VectorSubcoreMesh(core_axis_name="c", subcore_axis_name="s")
    k = pl.kernel(
        _sc_body,
        mesh=mesh,
        out_type=(
            jax.ShapeDtypeStruct((NC, ACC_ROWS, D), jnp.float32),
            jax.ShapeDtypeStruct((NC, DEG_ROWS, D), jnp.float32),
        ),
        scratch_types=[
            pltpu.VMEM_SHARED((ACC_ROWS, D), jnp.float32),
            pltpu.VMEM_SHARED((DEG_ROWS, D), jnp.float32),
            pltpu.VMEM((B,), jnp.int32),
            pltpu.VMEM((B,), jnp.int32),
            pltpu.VMEM((B,), jnp.int32),
            pltpu.VMEM((B,), jnp.int32),
            pltpu.VMEM((B,), jnp.int32),
            pltpu.VMEM((B,), jnp.int32),
            pltpu.VMEM((B,), jnp.float32),
            pltpu.VMEM((B,), jnp.float32),
            pltpu.VMEM((B,), jnp.float32),
            pltpu.VMEM((B,), jnp.float32),
            pltpu.VMEM((B,), jnp.float32),
            pltpu.VMEM((B,), jnp.float32),
            pltpu.VMEM((B, D), jnp.float32),
            pltpu.VMEM((B, D), jnp.float32),
            pltpu.VMEM((B, D), jnp.float32),
            pltpu.VMEM((B, D), jnp.float32),
            pltpu.VMEM((B,), jnp.int32),
            pltpu.VMEM((B,), jnp.int32),
            pltpu.VMEM((16, 16), jnp.float32),
            pltpu.SemaphoreType.DMA,
            pltpu.SemaphoreType.DMA,
            pltpu.SemaphoreType.DMA,
            pltpu.SemaphoreType.DMA,
            pltpu.SemaphoreType.DMA,
            pltpu.SemaphoreType.DMA,
            pltpu.SemaphoreType.DMA,
            pltpu.SemaphoreType.DMA,
            pltpu.SemaphoreType.DMA,
        ],
    )
    return k(src, dst, ew, em, feat)


BM = 1280


def _tc_finish_body(acc_ref, deg_ref, feat_ref, wst_ref, wnt_ref, b_ref, out_ref):
    msg = acc_ref[0] + acc_ref[1]
    deg = (deg_ref[0] + deg_ref[1]).reshape(BM, 1)
    h = msg / jnp.maximum(deg, 1.0)
    out_ref[...] = (
        jnp.dot(feat_ref[...], wst_ref[...], preferred_element_type=jnp.float32)
        + jnp.dot(h, wnt_ref[...], preferred_element_type=jnp.float32)
        + b_ref[...]
    )


def _tc_finish(acc, deg, feat, wst, wnt, b):
    grid = ((N_NODES + BM - 1) // BM,)
    return pl.pallas_call(
        _tc_finish_body,
        grid=grid,
        in_specs=[
            pl.BlockSpec((NC, BM, D), lambda i: (0, i, 0)),
            pl.BlockSpec((NC, BM), lambda i: (0, i)),
            pl.BlockSpec((BM, D), lambda i: (i, 0)),
            pl.BlockSpec((D, D), lambda i: (0, 0)),
            pl.BlockSpec((D, D), lambda i: (0, 0)),
            pl.BlockSpec((1, D), lambda i: (0, 0)),
        ],
        out_specs=pl.BlockSpec((BM, D), lambda i: (i, 0)),
        out_shape=jax.ShapeDtypeStruct((N_NODES, D), jnp.float32),
    )(acc, deg, feat, wst, wnt, b)


def kernel(feat, edge_index, edge_weight, edge_mask,
           W_self, b_self, W_neigh, b_neigh):
    src = edge_index[0].astype(jnp.int32)
    dst = edge_index[1].astype(jnp.int32)
    ew = edge_weight.reshape(-1)
    em = edge_mask.reshape(-1)
    acc, deg = _sc_aggregate(src, dst, ew, em, feat)
    deg = deg.reshape(NC, DEG_ROWS * D)
    b = (b_self + b_neigh).reshape(1, D)
    return _tc_finish(acc, deg, feat, W_self.T, W_neigh.T, b)


# speedup vs baseline: 9.2813x; 1.3601x over previous
import jax
import jax.numpy as jnp
from jax import lax
from jax.experimental import pallas as pl
from jax.experimental.pallas import tpu as pltpu
from jax.experimental.pallas import tpu_sc as plsc

N_NODES = 10000
N_EDGES = 320000
D = 128
NC = 2
NS = 16
NW = NC * NS
EPW = N_EDGES // NW
B = 80
NB = EPW // B
ROWS_PER_TILE = 632
ACC_ROWS = NS * ROWS_PER_TILE
DEG_ROWS = 80


def _sc_body(src_hbm, dst_hbm, ew_hbm, em_hbm, feat_hbm,
             out_hbm, outd_hbm,
             acc, accd,
             srcb0, srcb1, srcb2, dstb0, dstb1, dstb2,
             ewb0, ewb1, ewb2, emb0, emb1, emb2,
             gbuf0, gbuf1, dbuf0, dbuf1, dstd0, dstd1, idmat,
             gsem0, gsem1, isem0, isem1, isem2, ssem0, ssem1, tsem0, tsem1):
    c = lax.axis_index("c")
    s = lax.axis_index("s")
    wid = c * NS + s
    ebase = wid * EPW

    zeros16 = jnp.zeros((16,), jnp.float32)
    zeros16f = jnp.zeros((16,), jnp.float32)
    ones16 = jnp.ones((16,), jnp.float32)
    iota16 = lax.broadcasted_iota(jnp.int32, (16,), 0)
    gbufs = (gbuf0, gbuf1)
    gsems = (gsem0, gsem1)
    dbufs = (dbuf0, dbuf1)
    dstds = (dstd0, dstd1)
    ssems = (ssem0, ssem1)
    tsems = (tsem0, tsem1)
    isems = (isem0, isem1, isem2)
    srcbs = (srcb0, srcb1, srcb2)
    dstbs = (dstb0, dstb1, dstb2)
    ewbs = (ewb0, ewb1, ewb2)
    embs = (emb0, emb1, emb2)

    def _mk_id(k, carry):
        d = iota16 - jnp.full((16,), k, jnp.int32)
        idmat[k, pl.ds(0, 16)] = (1 - jnp.minimum(jnp.abs(d), 1)).astype(
            jnp.float32)
        return carry
    lax.fori_loop(0, 16, _mk_id, 0)

    def _zero_row(r, carry):
        for cc in range(D // 16):
            sl = pl.ds(cc * 16, 16)
            gbuf0[r, sl] = zeros16
            dbuf0[r, sl] = zeros16
            dbuf1[r, sl] = zeros16
        return carry
    lax.fori_loop(0, B, _zero_row, 0)

    tile_base = s * ROWS_PER_TILE
    for k in range(7):
        pltpu.sync_copy(gbuf0, acc.at[pl.ds(tile_base + k * B, B)])
    pltpu.sync_copy(gbuf0.at[pl.ds(0, 72)], acc.at[pl.ds(tile_base + 560, 72)])

    @pl.when(s == 0)
    def _zero_deg():
        pltpu.sync_copy(gbuf0, accd)

    plsc.subcore_barrier()

    def _issue_idx(i, b3):
        base = ebase + i * B
        pltpu.async_copy(src_hbm.at[pl.ds(base, B)], srcbs[b3], isems[b3])
        pltpu.async_copy(dst_hbm.at[pl.ds(base, B)], dstbs[b3], isems[b3])
        pltpu.async_copy(ew_hbm.at[pl.ds(base, B)], ewbs[b3], isems[b3])
        pltpu.async_copy(em_hbm.at[pl.ds(base, B)], embs[b3], isems[b3])

    def _drain_idx(i, b3):
        base = ebase + i * B
        pltpu.make_async_copy(src_hbm.at[pl.ds(base, B)], srcbs[b3], isems[b3]).wait()
        pltpu.make_async_copy(dst_hbm.at[pl.ds(base, B)], dstbs[b3], isems[b3]).wait()
        pltpu.make_async_copy(ew_hbm.at[pl.ds(base, B)], ewbs[b3], isems[b3]).wait()
        pltpu.make_async_copy(em_hbm.at[pl.ds(base, B)], embs[b3], isems[b3]).wait()

    def _wait_scatters(b2, b3):
        pltpu.make_async_copy(gbufs[b2], acc.at[dstbs[b3]], ssems[b2]).wait()
        pltpu.make_async_copy(dbufs[b2], accd.at[dstds[b2]], tsems[b2]).wait()

    def _compute(b2, b3):
        gbuf = gbufs[b2]
        dbuf = dbufs[b2]

        def _group(g, gcarry):
            sl16 = pl.ds(g * 16, 16)
            wv16 = ewbs[b3][sl16] * embs[b3][sl16]
            dv16 = dstbs[b3][sl16]
            dstds[b2][sl16] = lax.shift_right_logical(dv16, 7)
            plsc.store_scatter(dbuf, [iota16 + g * 16,
                                      lax.bitwise_and(dv16, 127)], ones16)
            for k in range(16):
                wb = jnp.full((16,), wv16[k], jnp.float32)
                r = g * 16 + k
                for cc in range(D // 16):
                    sl = pl.ds(cc * 16, 16)
                    gbuf[r, sl] = gbuf[r, sl] * wb
            return gcarry
        lax.fori_loop(0, B // 16, _group, 0)

    def _clear_deg(b2, b3):
        def _group(g, gcarry):
            dv16 = dstbs[b3][pl.ds(g * 16, 16)]
            plsc.store_scatter(dbufs[b2], [iota16 + g * 16,
                                           lax.bitwise_and(dv16, 127)],
                               zeros16f)
            return gcarry
        lax.fori_loop(0, B // 16, _group, 0)

    def _batch(i, k, first=False, last=False):
        b2, b3 = k % 2, k % 3
        pb2, pb3 = (k + 1) % 2, (k + 2) % 3
        pltpu.make_async_copy(feat_hbm.at[srcbs[b3]], gbufs[b2], gsems[b2]).wait()
        if not first:
            _wait_scatters(pb2, pb3)
            _clear_deg(pb2, pb3)
        if not last:
            _drain_idx(i + 1, (k + 1) % 3)
            pltpu.async_copy(feat_hbm.at[srcbs[(k + 1) % 3]], gbufs[pb2],
                             gsems[pb2])
        _compute(b2, b3)
        pltpu.async_copy(gbufs[b2], acc.at[dstbs[b3]], ssems[b2], add=True)
        pltpu.async_copy(dbufs[b2], accd.at[dstds[b2]], tsems[b2], add=True)
        if not last:
            @pl.when(i + 2 < NB)
            def _():
                _issue_idx(i + 2, pb3)

    _issue_idx(0, 0)
    _drain_idx(0, 0)
    _issue_idx(1, 1)
    pltpu.async_copy(feat_hbm.at[srcb0], gbuf0, gsem0)

    _batch(0, 0, first=True)
    for k in range(1, 6):
        _batch(k, k)

    def _six(p, carry):
        for k in range(6):
            _batch(6 * p + k, k)
        return carry
    lax.fori_loop(1, NB // 6, _six, 0)

    for k in range(4):
        _batch(120 + k, k)
    _batch(124, 4, last=True)
    _wait_scatters(0, 1)

    plsc.subcore_barrier()

    pltpu.sync_copy(acc.at[pl.ds(tile_base, ROWS_PER_TILE)],
                    out_hbm.at[c, pl.ds(tile_base, ROWS_PER_TILE)])

    @pl.when(s == 0)
    def _copy_deg():
        pltpu.sync_copy(accd, outd_hbm.at[c])


def _sc_aggregate(src, dst, ew, em, feat):
    mesh = plsc.VectorSubcoreMesh(core_axis_name="c", subcore_axis_name="s")
    k = pl.kernel(
        _sc_body,
        mesh=mesh,
        compiler_params=pltpu.CompilerParams(needs_layout_passes=False),
        out_type=(
            jax.ShapeDtypeStruct((NC, ACC_ROWS, D), jnp.float32),
            jax.ShapeDtypeStruct((NC, DEG_ROWS, D), jnp.float32),
        ),
        scratch_types=[
            pltpu.VMEM_SHARED((ACC_ROWS, D), jnp.float32),
            pltpu.VMEM_SHARED((DEG_ROWS, D), jnp.float32),
            pltpu.VMEM((B,), jnp.int32),
            pltpu.VMEM((B,), jnp.int32),
            pltpu.VMEM((B,), jnp.int32),
            pltpu.VMEM((B,), jnp.int32),
            pltpu.VMEM((B,), jnp.int32),
            pltpu.VMEM((B,), jnp.int32),
            pltpu.VMEM((B,), jnp.float32),
            pltpu.VMEM((B,), jnp.float32),
            pltpu.VMEM((B,), jnp.float32),
            pltpu.VMEM((B,), jnp.float32),
            pltpu.VMEM((B,), jnp.float32),
            pltpu.VMEM((B,), jnp.float32),
            pltpu.VMEM((B, D), jnp.float32),
            pltpu.VMEM((B, D), jnp.float32),
            pltpu.VMEM((B, D), jnp.float32),
            pltpu.VMEM((B, D), jnp.float32),
            pltpu.VMEM((B,), jnp.int32),
            pltpu.VMEM((B,), jnp.int32),
            pltpu.VMEM((16, 16), jnp.float32),
            pltpu.SemaphoreType.DMA,
            pltpu.SemaphoreType.DMA,
            pltpu.SemaphoreType.DMA,
            pltpu.SemaphoreType.DMA,
            pltpu.SemaphoreType.DMA,
            pltpu.SemaphoreType.DMA,
            pltpu.SemaphoreType.DMA,
            pltpu.SemaphoreType.DMA,
            pltpu.SemaphoreType.DMA,
        ],
    )
    return k(src, dst, ew, em, feat)


BM = 1280


def _tc_finish_body(acc_ref, deg_ref, feat_ref, wst_ref, wnt_ref, b_ref, out_ref):
    msg = acc_ref[0] + acc_ref[1]
    deg = (deg_ref[0] + deg_ref[1]).reshape(BM, 1)
    h = msg / jnp.maximum(deg, 1.0)
    out_ref[...] = (
        jnp.dot(feat_ref[...], wst_ref[...], preferred_element_type=jnp.float32)
        + jnp.dot(h, wnt_ref[...], preferred_element_type=jnp.float32)
        + b_ref[...]
    )


def _tc_finish(acc, deg, feat, wst, wnt, b):
    grid = ((N_NODES + BM - 1) // BM,)
    return pl.pallas_call(
        _tc_finish_body,
        grid=grid,
        in_specs=[
            pl.BlockSpec((NC, BM, D), lambda i: (0, i, 0)),
            pl.BlockSpec((NC, BM), lambda i: (0, i)),
            pl.BlockSpec((BM, D), lambda i: (i, 0)),
            pl.BlockSpec((D, D), lambda i: (0, 0)),
            pl.BlockSpec((D, D), lambda i: (0, 0)),
            pl.BlockSpec((1, D), lambda i: (0, 0)),
        ],
        out_specs=pl.BlockSpec((BM, D), lambda i: (i, 0)),
        out_shape=jax.ShapeDtypeStruct((N_NODES, D), jnp.float32),
    )(acc, deg, feat, wst, wnt, b)


def kernel(feat, edge_index, edge_weight, edge_mask,
           W_self, b_self, W_neigh, b_neigh):
    src = edge_index[0].astype(jnp.int32)
    dst = edge_index[1].astype(jnp.int32)
    ew = edge_weight.reshape(-1)
    em = edge_mask.reshape(-1)
    acc, deg = _sc_aggregate(src, dst, ew, em, feat)
    deg = deg.reshape(NC, DEG_ROWS * D)
    b = (b_self + b_neigh).reshape(1, D)
    return _tc_finish(acc, deg, feat, W_self.T, W_neigh.T, b)
